# baseline (speedup 1.0000x reference)
import jax
import jax.numpy as jnp
from jax import lax
from jax.experimental import pallas as pl

D = 1024
E = 8
TOP_K = 2
TN = 1024
KO = 4
DO = D // KO

_DN_T = (((1,), (1,)), ((), ()))


def _moe_block_kernel(x_ref, xo_ref, Ws_ref, bs_ref, Wr_ref, br_ref, Wg_ref,
                      bg_ref, gb_ref, out_ref):
    x = x_ref[...]

    scores = lax.dot_general(
        x, Wg_ref[...], _DN_T,
        preferred_element_type=jnp.float32) + bg_ref[...] + gb_ref[...]
    neg_inf = jnp.float32(-jnp.inf)
    v1 = jnp.max(scores, axis=-1, keepdims=True)
    eidx = lax.broadcasted_iota(jnp.int32, scores.shape, 1)
    a1 = jnp.min(jnp.where(scores == v1, eidx, E), axis=-1, keepdims=True)
    h1 = eidx == a1
    scores2 = jnp.where(h1, neg_inf, scores)
    v2 = jnp.max(scores2, axis=-1, keepdims=True)
    a2 = jnp.min(jnp.where(scores2 == v2, eidx, E), axis=-1, keepdims=True)
    h2 = eidx == a2
    w1 = jax.nn.sigmoid(v1 - v2)
    w2 = 1.0 - w1
    gates = h1 * w1 + h2 * w2

    acc = lax.dot_general(x, Ws_ref[...], _DN_T,
                          preferred_element_type=jnp.float32)
    acc = acc + bs_ref[...] + xo_ref[...]

    for e in range(E):
        ye = lax.dot_general(x, Wr_ref[e], _DN_T,
                             preferred_element_type=jnp.float32)
        acc = acc + gates[:, e:e + 1] * (ye + br_ref[e])

    out_ref[...] = jnp.maximum(acc, 0.0)


@jax.jit
def kernel(x, Ws, bs, Wr, br, Wg, bg, gate_bias):
    N = x.shape[0]
    bs2 = bs.reshape(1, D)
    br2 = br.reshape(E, 1, D)
    bg2 = bg.reshape(1, E)
    gb2 = gate_bias.reshape(1, E)

    grid = (KO, N // TN)
    out = pl.pallas_call(
        _moe_block_kernel,
        grid=grid,
        in_specs=[
            pl.BlockSpec((TN, D), lambda k, i: (i, 0)),
            pl.BlockSpec((TN, DO), lambda k, i: (i, k)),
            pl.BlockSpec((DO, D), lambda k, i: (k, 0)),
            pl.BlockSpec((1, DO), lambda k, i: (0, k)),
            pl.BlockSpec((E, DO, D), lambda k, i: (0, k, 0)),
            pl.BlockSpec((E, 1, DO), lambda k, i: (0, 0, k)),
            pl.BlockSpec((E, D), lambda k, i: (0, 0)),
            pl.BlockSpec((1, E), lambda k, i: (0, 0)),
            pl.BlockSpec((1, E), lambda k, i: (0, 0)),
        ],
        out_specs=pl.BlockSpec((TN, DO), lambda k, i: (i, k)),
        out_shape=jax.ShapeDtypeStruct((N, D), jnp.float32),
    )(x, x, Ws, bs2, Wr, br2, Wg, bg2, gb2)
    return out


# speedup vs baseline: 1.0646x; 1.0646x over previous
import jax
import jax.numpy as jnp
from jax import lax
from jax.experimental import pallas as pl
from jax.experimental.pallas import tpu as pltpu

D = 1024
E = 8
TOP_K = 2
TN = 1024
KO = 2
DO = D // KO

_DN_T = (((1,), (1,)), ((), ()))


def _moe_block_kernel(x_ref, Ws_ref, bs_ref, Wr_ref, br_ref, Wg_ref,
                      bg_ref, gb_ref, out_ref, gates_ref):
    k = pl.program_id(0)
    i = pl.program_id(1)
    x = x_ref[...]

    @pl.when(k == 0)
    def _gating():
        scores = lax.dot_general(
            x, Wg_ref[...], _DN_T,
            preferred_element_type=jnp.float32) + bg_ref[...] + gb_ref[...]
        neg_inf = jnp.float32(-jnp.inf)
        v1 = jnp.max(scores, axis=-1, keepdims=True)
        eidx = lax.broadcasted_iota(jnp.int32, scores.shape, 1)
        a1 = jnp.min(jnp.where(scores == v1, eidx, E), axis=-1, keepdims=True)
        h1 = eidx == a1
        scores2 = jnp.where(h1, neg_inf, scores)
        v2 = jnp.max(scores2, axis=-1, keepdims=True)
        a2 = jnp.min(jnp.where(scores2 == v2, eidx, E), axis=-1, keepdims=True)
        h2 = eidx == a2
        w1 = jax.nn.sigmoid(v1 - v2)
        gates_ref[i] = h1 * w1 + h2 * (1.0 - w1)

    gates = gates_ref[i]

    acc = lax.dot_general(x, Ws_ref[...], _DN_T,
                          preferred_element_type=jnp.float32)
    xo = jnp.where(k == 0, x[:, :DO], x[:, DO:])
    acc = acc + bs_ref[...] + xo

    for e in range(E):
        ye = lax.dot_general(x, Wr_ref[e], _DN_T,
                             preferred_element_type=jnp.float32)
        acc = acc + gates[:, e:e + 1] * (ye + br_ref[e])

    out_ref[...] = jnp.maximum(acc, 0.0)


@jax.jit
def kernel(x, Ws, bs, Wr, br, Wg, bg, gate_bias):
    N = x.shape[0]
    bs2 = bs.reshape(1, D)
    br2 = br.reshape(E, 1, D)
    bg2 = bg.reshape(1, E)
    gb2 = gate_bias.reshape(1, E)

    grid = (KO, N // TN)
    out = pl.pallas_call(
        _moe_block_kernel,
        grid=grid,
        in_specs=[
            pl.BlockSpec((TN, D), lambda k, i: (i, 0)),
            pl.BlockSpec((DO, D), lambda k, i: (k, 0)),
            pl.BlockSpec((1, DO), lambda k, i: (0, k)),
            pl.BlockSpec((E, DO, D), lambda k, i: (0, k, 0)),
            pl.BlockSpec((E, 1, DO), lambda k, i: (0, 0, k)),
            pl.BlockSpec((E, D), lambda k, i: (0, 0)),
            pl.BlockSpec((1, E), lambda k, i: (0, 0)),
            pl.BlockSpec((1, E), lambda k, i: (0, 0)),
        ],
        out_specs=pl.BlockSpec((TN, DO), lambda k, i: (i, k)),
        out_shape=jax.ShapeDtypeStruct((N, D), jnp.float32),
        scratch_shapes=[pltpu.VMEM((N // TN, TN, E), jnp.float32)],
    )(x, Ws, bs2, Wr, br2, Wg, bg2, gb2)
    return out


# speedup vs baseline: 1.0768x; 1.0115x over previous
import jax
import jax.numpy as jnp
from jax import lax
from jax.experimental import pallas as pl

D = 1024
E = 8
TOP_K = 2
TN = 1024
KO = 2
DO = D // KO

_DN_T = (((1,), (1,)), ((), ()))


def _moe_block_kernel(x_ref, xo_ref, Ws_ref, bs_ref, Wr_ref, br_ref, Wg_ref,
                      bg_ref, gb_ref, out_ref):
    x = x_ref[...]

    scores = lax.dot_general(
        x, Wg_ref[...], _DN_T,
        preferred_element_type=jnp.float32) + bg_ref[...] + gb_ref[...]
    neg_inf = jnp.float32(-jnp.inf)
    v1 = jnp.max(scores, axis=-1, keepdims=True)
    eidx = lax.broadcasted_iota(jnp.int32, scores.shape, 1)
    a1 = jnp.min(jnp.where(scores == v1, eidx, E), axis=-1, keepdims=True)
    h1 = eidx == a1
    scores2 = jnp.where(h1, neg_inf, scores)
    v2 = jnp.max(scores2, axis=-1, keepdims=True)
    a2 = jnp.min(jnp.where(scores2 == v2, eidx, E), axis=-1, keepdims=True)
    h2 = eidx == a2
    w1 = jax.nn.sigmoid(v1 - v2)
    w2 = 1.0 - w1
    gates = h1 * w1 + h2 * w2

    acc = lax.dot_general(x, Ws_ref[...], _DN_T,
                          preferred_element_type=jnp.float32)
    acc = acc + bs_ref[...] + xo_ref[...]

    for e in range(E):
        ye = lax.dot_general(x, Wr_ref[e], _DN_T,
                             preferred_element_type=jnp.float32)
        acc = acc + gates[:, e:e + 1] * (ye + br_ref[e])

    out_ref[...] = jnp.maximum(acc, 0.0)


@jax.jit
def kernel(x, Ws, bs, Wr, br, Wg, bg, gate_bias):
    N = x.shape[0]
    bs2 = bs.reshape(1, D)
    br2 = br.reshape(E, 1, D)
    bg2 = bg.reshape(1, E)
    gb2 = gate_bias.reshape(1, E)

    grid = (KO, N // TN)
    out = pl.pallas_call(
        _moe_block_kernel,
        grid=grid,
        in_specs=[
            pl.BlockSpec((TN, D), lambda k, i: (i, 0)),
            pl.BlockSpec((TN, DO), lambda k, i: (i, k)),
            pl.BlockSpec((DO, D), lambda k, i: (k, 0)),
            pl.BlockSpec((1, DO), lambda k, i: (0, k)),
            pl.BlockSpec((E, DO, D), lambda k, i: (0, k, 0)),
            pl.BlockSpec((E, 1, DO), lambda k, i: (0, 0, k)),
            pl.BlockSpec((E, D), lambda k, i: (0, 0)),
            pl.BlockSpec((1, E), lambda k, i: (0, 0)),
            pl.BlockSpec((1, E), lambda k, i: (0, 0)),
        ],
        out_specs=pl.BlockSpec((TN, DO), lambda k, i: (i, k)),
        out_shape=jax.ShapeDtypeStruct((N, D), jnp.float32),
    )(x, x, Ws, bs2, Wr, br2, Wg, bg2, gb2)
    return out
